# baseline (speedup 1.0000x reference)
import functools

import numpy as np
import jax
import jax.numpy as jnp
from jax import lax
from jax.experimental import pallas as pl
from jax.experimental.pallas import tpu as pltpu
from jax.experimental.pallas import tpu_sc as plsc

_L = 128
_V = 16


def _nll_body(conf_ref, tgt_ref, nll_ref, *, C):
    x = conf_ref[...][0]
    xt = x.T
    E = jnp.exp(xt)
    sums = jnp.sum(E, axis=0, keepdims=True)
    t = tgt_ref[...][0]
    riota = jax.lax.broadcasted_iota(jnp.int32, xt.shape, 0)
    g = jnp.sum(jnp.where(riota == t, xt, 0.0), axis=0, keepdims=True)
    nll_ref[...] = (jnp.log(sums) - g)[None]


def _sl1_body(r4_ref, d_ref, tgt_ref, out_ref):
    d = d_ref[...][0]
    a = jnp.abs(d)
    sl1 = jnp.where(a < 1.0, 0.5 * d * d, a - 0.5)
    posf = jnp.dot((tgt_ref[...][0] > 0).astype(jnp.float32), r4_ref[...],
                   preferred_element_type=jnp.float32)
    s = jnp.sum(sl1 * posf)
    lane0 = jax.lax.broadcasted_iota(jnp.int32, (1, _L), 1) == 0
    out_ref[...] = jnp.where(lane0, s, 0.0)[None]


def _sc_select_body(A, CH, nll_hbm, ct_hbm, out_hbm,
                    nll_v, ct_v, lossc_v, vec_v):
    nc = 2
    wid = lax.axis_index("s") * nc + lax.axis_index("c")
    nchunk = A // CH
    nvec = CH // _V

    def chunk_body(j, carry):
        pltpu.sync_copy(nll_hbm.at[wid, pl.ds(j * CH, CH)], nll_v)
        pltpu.sync_copy(ct_hbm.at[wid, pl.ds(j * CH, CH)], ct_v)

        def vec_body(i, c2):
            s0, g0, p, pn = c2
            v = nll_v[pl.ds(i * _V, _V)]
            t = ct_v[pl.ds(i * _V, _V)]
            pos = t > 0
            l = jnp.where(pos, 0.0, jnp.maximum(v, 0.0))
            lossc_v[pl.ds(j * CH + i * _V, _V)] = l
            one = jnp.ones((_V,), jnp.int32)
            zero = jnp.zeros((_V,), jnp.int32)
            return (s0 + l,
                    g0 + jnp.where(l > 0.0, one, zero),
                    p + jnp.where(pos, one, zero),
                    pn + jnp.where(pos, v, jnp.zeros((_V,), jnp.float32)))

        return lax.fori_loop(0, nvec, vec_body, carry)

    z_f = jnp.zeros((_V,), jnp.float32)
    z_i = jnp.zeros((_V,), jnp.int32)
    s0v, g0v, pv, pnv = lax.fori_loop(0, nchunk, chunk_body,
                                      (z_f, z_i, z_i, z_f))
    s0 = jnp.sum(s0v)
    g0 = jnp.sum(g0v)
    p = jnp.sum(pv)
    pn = jnp.sum(pnv)
    k = jnp.minimum(3 * p, A - 1)

    def easy_fn(_):
        return 0, g0, s0

    def hard_fn(_):
        def it(_, c2):
            lo, hi = c2
            mid = lo + ((hi - lo) >> 1)

            def cnt_body(i, acc):
                bits = plsc.bitcast(lossc_v[pl.ds(i * _V, _V)], jnp.int32)
                return acc + jnp.where(bits >= mid,
                                       jnp.ones((_V,), jnp.int32),
                                       jnp.zeros((_V,), jnp.int32))

            cnt = jnp.sum(lax.fori_loop(0, A // _V, cnt_body, z_i))
            big = cnt >= k
            return lax.select(big, mid, lo), lax.select(big, hi, mid)

        lo, _ = lax.fori_loop(0, 31, it, (0, 0x7F800001))

        def gs_body(i, c2):
            gacc, sacc = c2
            v = lossc_v[pl.ds(i * _V, _V)]
            bits = plsc.bitcast(v, jnp.int32)
            gt = bits > lo
            return (gacc + jnp.where(gt, jnp.ones((_V,), jnp.int32),
                                     jnp.zeros((_V,), jnp.int32)),
                    sacc + jnp.where(gt, v, z_f))

        gv, s1v = lax.fori_loop(0, A // _V, gs_body, (z_i, z_f))
        return lo, jnp.sum(gv), jnp.sum(s1v)

    tbits, G, S1 = lax.cond(k >= g0, easy_fn, hard_fn, None)
    tvec = plsc.bitcast(jnp.broadcast_to(tbits, (_V,)), jnp.float32)
    vec_v[...] = tvec
    T = vec_v[...][0]
    m = (k - G).astype(jnp.float32)
    negsum = S1 + jnp.where(m > 0, m * T, 0.0)
    row_conf = pn + negsum
    io = lax.broadcasted_iota(jnp.int32, (_V,), 0)
    vec_v[...] = (jnp.where(io == 0, row_conf, 0.0)
                  + jnp.where(io == 1, p.astype(jnp.float32), 0.0))
    pltpu.sync_copy(vec_v, out_hbm.at[pl.ds(wid * _V, _V)])


def kernel(loc, conf, loc_target, conf_target):
    b, A, C = conf.shape
    assert b == 32 and A % 2000 == 0
    ct = conf_target.astype(jnp.int32)
    ct3 = ct.reshape(b, 1, A)

    nll = pl.pallas_call(
        functools.partial(_nll_body, C=C),
        grid=(b,),
        in_specs=[
            pl.BlockSpec((1, A, C), lambda i: (i, 0, 0)),
            pl.BlockSpec((1, 1, A), lambda i: (i, 0, 0)),
        ],
        out_specs=pl.BlockSpec((1, 1, A), lambda i: (i, 0, 0)),
        out_shape=jax.ShapeDtypeStruct((b, 1, A), jnp.float32),
    )(conf, ct3)

    LB = (A * 4) // _L
    AW = _L // 4
    d2 = (loc - loc_target).reshape(b, LB, _L)
    ct_loc = ct.reshape(b, LB, AW)
    R4 = jnp.asarray(np.repeat(np.eye(AW, dtype=np.float32), 4, axis=1))

    sl1p = pl.pallas_call(
        _sl1_body,
        grid=(b,),
        in_specs=[
            pl.BlockSpec((AW, _L), lambda i: (0, 0)),
            pl.BlockSpec((1, LB, _L), lambda i: (i, 0, 0)),
            pl.BlockSpec((1, LB, AW), lambda i: (i, 0, 0)),
        ],
        out_specs=pl.BlockSpec((1, 1, _L), lambda i: (i, 0, 0)),
        out_shape=jax.ShapeDtypeStruct((b, 1, _L), jnp.float32),
    )(R4, d2, ct_loc)

    CH = 2000
    sel = pl.kernel(
        functools.partial(_sc_select_body, A, CH),
        out_type=jax.ShapeDtypeStruct((b * _V,), jnp.float32),
        mesh=plsc.VectorSubcoreMesh(core_axis_name="c", subcore_axis_name="s"),
        scratch_types=[
            pltpu.VMEM((CH,), jnp.float32),
            pltpu.VMEM((CH,), jnp.int32),
            pltpu.VMEM((A,), jnp.float32),
            pltpu.VMEM((_V,), jnp.float32),
        ],
        compiler_params=pltpu.CompilerParams(use_tc_tiling_on_sc=False,
                                             needs_layout_passes=False),
    )
    part = sel(nll.reshape(b, A), ct).reshape(b, _V)
    nf = jnp.sum(part[:, 1])
    lc_v = jnp.sum(part[:, 0]) / nf
    ll_v = jnp.sum(sl1p) / nf
    return (lc_v.reshape(()), ll_v.reshape(()), (lc_v + ll_v).reshape(()))


# speedup vs baseline: 1.0012x; 1.0012x over previous
import functools

import numpy as np
import jax
import jax.numpy as jnp
from jax import lax
from jax.experimental import pallas as pl
from jax.experimental.pallas import tpu as pltpu
from jax.experimental.pallas import tpu_sc as plsc

_L = 128
_V = 16


def _nll_body(conf_ref, tgt_ref, nll_ref, *, C):
    x = conf_ref[...][0]
    xt = x.T
    E = jnp.exp(xt)
    sums = jnp.sum(E, axis=0, keepdims=True)
    t = tgt_ref[...][0]
    riota = jax.lax.broadcasted_iota(jnp.int32, xt.shape, 0)
    g = jnp.sum(jnp.where(riota == t, xt, 0.0), axis=0, keepdims=True)
    nll_ref[...] = (jnp.log(sums) - g)[None]


def _sl1_body(r4_ref, d_ref, tgt_ref, out_ref):
    d = d_ref[...][0]
    a = jnp.abs(d)
    sl1 = jnp.where(a < 1.0, 0.5 * d * d, a - 0.5)
    posf = jnp.dot((tgt_ref[...][0] > 0).astype(jnp.float32), r4_ref[...],
                   preferred_element_type=jnp.float32)
    s = jnp.sum(sl1 * posf)
    lane0 = jax.lax.broadcasted_iota(jnp.int32, (1, _L), 1) == 0
    out_ref[...] = jnp.where(lane0, s, 0.0)[None]


def _sc_select_body(A, CH, nll_hbm, ct_hbm, out_hbm,
                    nll_v, ct_v, lossc_v, vec_v):
    nc = 2
    wid = lax.axis_index("s") * nc + lax.axis_index("c")
    nchunk = A // CH
    nvec = CH // _V

    def chunk_body(j, carry):
        pltpu.sync_copy(nll_hbm.at[wid, pl.ds(j * CH, CH)], nll_v)
        pltpu.sync_copy(ct_hbm.at[wid, pl.ds(j * CH, CH)], ct_v)

        def vec_body(i, c2):
            s0, g0, p, pn = c2
            v = nll_v[pl.ds(i * _V, _V)]
            t = ct_v[pl.ds(i * _V, _V)]
            pos = t > 0
            l = jnp.where(pos, 0.0, jnp.maximum(v, 0.0))
            lossc_v[pl.ds(j * CH + i * _V, _V)] = l
            one = jnp.ones((_V,), jnp.int32)
            zero = jnp.zeros((_V,), jnp.int32)
            return (s0 + l,
                    g0 + jnp.where(l > 0.0, one, zero),
                    p + jnp.where(pos, one, zero),
                    pn + jnp.where(pos, v, jnp.zeros((_V,), jnp.float32)))

        return lax.fori_loop(0, nvec, vec_body, carry)

    z_f = jnp.zeros((_V,), jnp.float32)
    z_i = jnp.zeros((_V,), jnp.int32)
    s0v, g0v, pv, pnv = lax.fori_loop(0, nchunk, chunk_body,
                                      (z_f, z_i, z_i, z_f))
    s0 = jnp.sum(s0v)
    g0 = jnp.sum(g0v)
    p = jnp.sum(pv)
    pn = jnp.sum(pnv)
    k = jnp.minimum(3 * p, A - 1)

    def easy_fn(_):
        return 0, g0, s0

    def hard_fn(_):
        def it(_, c2):
            lo, hi = c2
            mid = lo + ((hi - lo) >> 1)

            def cnt_body(i, acc):
                bits = plsc.bitcast(lossc_v[pl.ds(i * _V, _V)], jnp.int32)
                return acc + jnp.where(bits >= mid,
                                       jnp.ones((_V,), jnp.int32),
                                       jnp.zeros((_V,), jnp.int32))

            cnt = jnp.sum(lax.fori_loop(0, A // _V, cnt_body, z_i))
            big = cnt >= k
            return lax.select(big, mid, lo), lax.select(big, hi, mid)

        lo, _ = lax.fori_loop(0, 31, it, (0, 0x7F800001))

        def gs_body(i, c2):
            gacc, sacc = c2
            v = lossc_v[pl.ds(i * _V, _V)]
            bits = plsc.bitcast(v, jnp.int32)
            gt = bits > lo
            return (gacc + jnp.where(gt, jnp.ones((_V,), jnp.int32),
                                     jnp.zeros((_V,), jnp.int32)),
                    sacc + jnp.where(gt, v, z_f))

        gv, s1v = lax.fori_loop(0, A // _V, gs_body, (z_i, z_f))
        return lo, jnp.sum(gv), jnp.sum(s1v)

    tbits, G, S1 = lax.cond(k >= g0, easy_fn, hard_fn, None)
    tvec = plsc.bitcast(jnp.broadcast_to(tbits, (_V,)), jnp.float32)
    vec_v[...] = tvec
    T = vec_v[...][0]
    m = (k - G).astype(jnp.float32)
    negsum = S1 + jnp.where(m > 0, m * T, 0.0)
    row_conf = pn + negsum
    io = lax.broadcasted_iota(jnp.int32, (_V,), 0)
    vec_v[...] = (jnp.where(io == 0, row_conf, 0.0)
                  + jnp.where(io == 1, p.astype(jnp.float32), 0.0))
    pltpu.sync_copy(vec_v, out_hbm.at[pl.ds(wid * _V, _V)])


def kernel(loc, conf, loc_target, conf_target):
    b, A, C = conf.shape
    assert b == 32 and A % 2000 == 0
    ct = conf_target.astype(jnp.int32)
    ct3 = ct.reshape(b, 1, A)

    nll = pl.pallas_call(
        functools.partial(_nll_body, C=C),
        grid=(b,),
        in_specs=[
            pl.BlockSpec((1, A, C), lambda i: (i, 0, 0)),
            pl.BlockSpec((1, 1, A), lambda i: (i, 0, 0)),
        ],
        out_specs=pl.BlockSpec((1, 1, A), lambda i: (i, 0, 0)),
        out_shape=jax.ShapeDtypeStruct((b, 1, A), jnp.float32),
    )(conf, ct3)

    LB = (A * 4) // _L
    AW = _L // 4
    d2 = loc.reshape(b, LB, _L) - loc_target.reshape(b, LB, _L)
    ct_loc = ct.reshape(b, LB, AW)
    R4 = jnp.asarray(np.repeat(np.eye(AW, dtype=np.float32), 4, axis=1))

    sl1p = pl.pallas_call(
        _sl1_body,
        grid=(b,),
        in_specs=[
            pl.BlockSpec((AW, _L), lambda i: (0, 0)),
            pl.BlockSpec((1, LB, _L), lambda i: (i, 0, 0)),
            pl.BlockSpec((1, LB, AW), lambda i: (i, 0, 0)),
        ],
        out_specs=pl.BlockSpec((1, 1, _L), lambda i: (i, 0, 0)),
        out_shape=jax.ShapeDtypeStruct((b, 1, _L), jnp.float32),
    )(R4, d2, ct_loc)

    CH = 2000
    sel = pl.kernel(
        functools.partial(_sc_select_body, A, CH),
        out_type=jax.ShapeDtypeStruct((b * _V,), jnp.float32),
        mesh=plsc.VectorSubcoreMesh(core_axis_name="c", subcore_axis_name="s"),
        scratch_types=[
            pltpu.VMEM((CH,), jnp.float32),
            pltpu.VMEM((CH,), jnp.int32),
            pltpu.VMEM((A,), jnp.float32),
            pltpu.VMEM((_V,), jnp.float32),
        ],
        compiler_params=pltpu.CompilerParams(use_tc_tiling_on_sc=False,
                                             needs_layout_passes=False),
    )
    part = sel(nll.reshape(b, A), ct).reshape(b, _V)
    nf = jnp.sum(part[:, 1])
    lc_v = jnp.sum(part[:, 0]) / nf
    ll_v = jnp.sum(sl1p) / nf
    return (lc_v.reshape(()), ll_v.reshape(()), (lc_v + ll_v).reshape(()))
